# baseline (speedup 1.0000x reference)
import functools

import jax
import jax.numpy as jnp
from jax import lax
from jax.experimental import pallas as pl
from jax.experimental.pallas import tpu as pltpu
from jax.experimental.pallas import tpu_sc as plsc

B = 16384
ND = 13
NS = 26
V = 100000
D = 16

_info = plsc.get_sparse_core_info()
NC = _info.num_cores
NSUB = _info.num_subcores
L = _info.num_lanes
NW = NC * NSUB
ROWS_W = B // NW
E_W = ROWS_W * NS
CHUNK = 128
NCHUNK = ROWS_W // CHUNK
GROUP = 128
GPC = CHUNK * NS // GROUP


def _sc_pool_body(idxf_hbm, table_hbm, fm_hbm, idx_f, idx_i, rows, out_v, sem):
    wid = lax.axis_index("s") * NC + lax.axis_index("c")
    base = wid * ROWS_W
    ebase = wid * E_W

    pltpu.sync_copy(idxf_hbm.at[pl.ds(ebase, E_W)], idx_f)

    def conv_body(j, carry):
        p = j * L
        v = idx_f[pl.ds(p, L)].astype(jnp.int32)
        fld = (p + lax.broadcasted_iota(jnp.int32, (L,), 0)) % NS
        idx_i[pl.ds(p, L)] = v + fld * V
        return carry

    lax.fori_loop(0, E_W // L, conv_body, None)

    for c in range(NCHUNK):
        cps = []
        for g in range(GPC):
            cp = pltpu.async_copy(
                table_hbm.at[idx_i.at[pl.ds(c * CHUNK * NS + g * GROUP, GROUP)]],
                rows.at[pl.ds(g * GROUP, GROUP)],
                sem,
            )
            cps.append(cp)
        for cp in cps:
            cp.wait()

        def row_body(r, carry):
            j0 = r * NS
            e = rows[j0, :]
            s = e
            sq = e * e
            for i in range(1, NS):
                e = rows[j0 + i, :]
                s = s + e
                sq = sq + e * e
            out_v[c * CHUNK + r, :] = 0.5 * (s * s - sq)
            return carry

        lax.fori_loop(0, CHUNK, row_body, None)

    pltpu.sync_copy(out_v, fm_hbm.at[pl.ds(base, ROWS_W)])


_sc_pool = pl.kernel(
    _sc_pool_body,
    out_type=jax.ShapeDtypeStruct((B, D), jnp.float32),
    mesh=plsc.VectorSubcoreMesh(core_axis_name="c", subcore_axis_name="s"),
    scratch_types=[
        pltpu.VMEM((E_W,), jnp.float32),
        pltpu.VMEM((E_W,), jnp.int32),
        pltpu.VMEM((CHUNK * NS, D), jnp.float32),
        pltpu.VMEM((ROWS_W, D), jnp.float32),
        pltpu.SemaphoreType.DMA,
    ],
    compiler_params=pltpu.CompilerParams(use_tc_tiling_on_sc=False),
)


BT = 1024


def _mlp_body(inp_ref, fm_ref, gamma_ref, beta_ref, mean_ref, var_ref,
              w1_ref, b1_ref, w2_ref, b2_ref, w3_ref, b3_ref, wo_ref, bo_ref,
              out_ref):
    x = jnp.concatenate([inp_ref[:, :ND], fm_ref[...]], axis=1)
    scale = gamma_ref[...] * lax.rsqrt(var_ref[...] + 1e-3)
    x = (x - mean_ref[...]) * scale + beta_ref[...]
    h = jnp.maximum(
        jnp.dot(x, w1_ref[...], preferred_element_type=jnp.float32) + b1_ref[...], 0.0)
    h = jnp.maximum(
        jnp.dot(h, w2_ref[...], preferred_element_type=jnp.float32) + b2_ref[...], 0.0)
    h = jnp.maximum(
        jnp.dot(h, w3_ref[...], preferred_element_type=jnp.float32) + b3_ref[...], 0.0)
    o = jnp.dot(h, wo_ref[...], preferred_element_type=jnp.float32) + bo_ref[...]
    out_ref[...] = jax.nn.sigmoid(o)


def _full(shape):
    return pl.BlockSpec(shape, lambda i: tuple(0 for _ in shape))


_mlp = pl.pallas_call(
    _mlp_body,
    grid=(B // BT,),
    in_specs=[
        pl.BlockSpec((BT, ND + NS), lambda i: (i, 0)),
        pl.BlockSpec((BT, D), lambda i: (i, 0)),
        _full((ND + D,)), _full((ND + D,)), _full((ND + D,)), _full((ND + D,)),
        _full((ND + D, 256)), _full((256,)),
        _full((256, 128)), _full((128,)),
        _full((128, 64)), _full((64,)),
        _full((64, 1)), _full((1,)),
    ],
    out_specs=pl.BlockSpec((BT, 1), lambda i: (i, 0)),
    out_shape=jax.ShapeDtypeStruct((B, 1), jnp.float32),
)


def kernel(inputs, tables, gamma, beta, moving_mean, moving_var,
           W1, b1, W2, b2, W3, b3, Wo, bo):
    idx_f = inputs[:, ND:].reshape(-1)
    table_flat = tables.reshape(NS * V, D)
    fm = _sc_pool(idx_f, table_flat)
    return _mlp(inputs, fm, gamma, beta, moving_mean, moving_var,
                W1, b1, W2, b2, W3, b3, Wo, bo)


# speedup vs baseline: 1.0010x; 1.0010x over previous
import functools

import jax
import jax.numpy as jnp
from jax import lax
from jax.experimental import pallas as pl
from jax.experimental.pallas import tpu as pltpu
from jax.experimental.pallas import tpu_sc as plsc

B = 16384
ND = 13
NS = 26
V = 100000
D = 16
NF = ND + NS

_info = plsc.get_sparse_core_info()
NC = _info.num_cores
NSUB = _info.num_subcores
L = _info.num_lanes
NW = NC * NSUB
ROWS_W = B // NW
CHUNK = 128
NCHUNK = ROWS_W // CHUNK


def _sc_pool_body(inputs_hbm, tables_hbm, fm_hbm, inp_v, idx_fm, rows, out_v, sem):
    wid = lax.axis_index("s") * NC + lax.axis_index("c")
    base = wid * ROWS_W

    pltpu.sync_copy(inputs_hbm.at[pl.ds(base, ROWS_W)], inp_v)

    lanes = lax.broadcasted_iota(jnp.int32, (L,), 0)
    off_a = lanes * ROWS_W
    off_b = (lanes + (NS - L)) * ROWS_W

    def trans_body(r, carry):
        a = inp_v[r, pl.ds(ND, L)].astype(jnp.int32)
        b = inp_v[r, pl.ds(NF - L, L)].astype(jnp.int32)
        plsc.store_scatter(idx_fm, [off_a + r], a)
        plsc.store_scatter(idx_fm, [off_b + r], b)
        return carry

    lax.fori_loop(0, ROWS_W, trans_body, None)

    for c in range(NCHUNK):
        cps = []
        for f in range(NS):
            cp = pltpu.async_copy(
                tables_hbm.at[f].at[idx_fm.at[pl.ds(f * ROWS_W + c * CHUNK, CHUNK)]],
                rows.at[pl.ds(f * CHUNK, CHUNK)],
                sem,
            )
            cps.append(cp)
        for cp in cps:
            cp.wait()

        def row_body(k, carry):
            e = rows[k, :]
            s = e
            sq = e * e
            for f in range(1, NS):
                e = rows[f * CHUNK + k, :]
                s = s + e
                sq = sq + e * e
            out_v[c * CHUNK + k, :] = 0.5 * (s * s - sq)
            return carry

        lax.fori_loop(0, CHUNK, row_body, None)

    pltpu.sync_copy(out_v, fm_hbm.at[pl.ds(base, ROWS_W)])


_sc_pool = pl.kernel(
    _sc_pool_body,
    out_type=jax.ShapeDtypeStruct((B, D), jnp.float32),
    mesh=plsc.VectorSubcoreMesh(core_axis_name="c", subcore_axis_name="s"),
    scratch_types=[
        pltpu.VMEM((ROWS_W, NF), jnp.float32),
        pltpu.VMEM((NS * ROWS_W,), jnp.int32),
        pltpu.VMEM((CHUNK * NS, D), jnp.float32),
        pltpu.VMEM((ROWS_W, D), jnp.float32),
        pltpu.SemaphoreType.DMA,
    ],
    compiler_params=pltpu.CompilerParams(use_tc_tiling_on_sc=False,
                                         needs_layout_passes=False),
)


BT = 1024


def _mlp_body(inp_ref, fm_ref, gamma_ref, beta_ref, mean_ref, var_ref,
              w1_ref, b1_ref, w2_ref, b2_ref, w3_ref, b3_ref, wo_ref, bo_ref,
              out_ref):
    x = jnp.concatenate([inp_ref[:, :ND], fm_ref[...]], axis=1)
    scale = gamma_ref[...] * lax.rsqrt(var_ref[...] + 1e-3)
    x = (x - mean_ref[...]) * scale + beta_ref[...]
    h = jnp.maximum(
        jnp.dot(x, w1_ref[...], preferred_element_type=jnp.float32) + b1_ref[...], 0.0)
    h = jnp.maximum(
        jnp.dot(h, w2_ref[...], preferred_element_type=jnp.float32) + b2_ref[...], 0.0)
    h = jnp.maximum(
        jnp.dot(h, w3_ref[...], preferred_element_type=jnp.float32) + b3_ref[...], 0.0)
    o = jnp.dot(h, wo_ref[...], preferred_element_type=jnp.float32) + bo_ref[...]
    out_ref[...] = jax.nn.sigmoid(o)


def _full(shape):
    return pl.BlockSpec(shape, lambda i: tuple(0 for _ in shape))


_mlp = pl.pallas_call(
    _mlp_body,
    grid=(B // BT,),
    in_specs=[
        pl.BlockSpec((BT, NF), lambda i: (i, 0)),
        pl.BlockSpec((BT, D), lambda i: (i, 0)),
        _full((ND + D,)), _full((ND + D,)), _full((ND + D,)), _full((ND + D,)),
        _full((ND + D, 256)), _full((256,)),
        _full((256, 128)), _full((128,)),
        _full((128, 64)), _full((64,)),
        _full((64, 1)), _full((1,)),
    ],
    out_specs=pl.BlockSpec((BT, 1), lambda i: (i, 0)),
    out_shape=jax.ShapeDtypeStruct((B, 1), jnp.float32),
)


def kernel(inputs, tables, gamma, beta, moving_mean, moving_var,
           W1, b1, W2, b2, W3, b3, Wo, bo):
    fm = _sc_pool(inputs, tables)
    return _mlp(inputs, fm, gamma, beta, moving_mean, moving_var,
                W1, b1, W2, b2, W3, b3, Wo, bo)
